# baseline (speedup 1.0000x reference)
import functools

import jax
import jax.numpy as jnp
from jax import lax
from jax.experimental import pallas as pl
from jax.experimental.pallas import tpu as pltpu
from jax.experimental.pallas import tpu_sc as plsc

VOCAB = 100000
EMBED_DIM = 64
BATCH = 4096
HIST = 50
N = BATCH * HIST

NUM_CORES = 2
NUM_SUBCORES = 16
NW = NUM_CORES * NUM_SUBCORES
B_PER_W = BATCH // NW
CHUNK = B_PER_W
NCHUNK = HIST
NBUF = 8

_mesh = plsc.VectorSubcoreMesh(core_axis_name="c", subcore_axis_name="s")


@functools.partial(
    pl.kernel,
    mesh=_mesh,
    out_type=jax.ShapeDtypeStruct((NW, B_PER_W, HIST, EMBED_DIM),
                                  jnp.float32),
    compiler_params=pltpu.CompilerParams(use_tc_tiling_on_sc=False),
    scratch_types=[
        pltpu.VMEM((NCHUNK, CHUNK), jnp.int32),
        pltpu.VMEM((NBUF, CHUNK, EMBED_DIM), jnp.float32),
    ] + [pltpu.SemaphoreType.DMA] * (2 * NBUF),
)
def _emb_lookup(x_hbm, table_hbm, out_hbm, idx_v, rows_v, *sems):
    wid = lax.axis_index("s") * NUM_CORES + lax.axis_index("c")

    pltpu.sync_copy(x_hbm.at[:, wid], idx_v)

    gsems = sems[:NBUF]
    wsems = sems[NBUF:]

    def gather(j, b):
        pltpu.async_copy(table_hbm.at[idx_v.at[j]], rows_v.at[b], gsems[b])

    for b in range(NBUF):
        gather(b, b)

    def chunk_body(j, _):
        for b in range(NBUF):
            @pl.when(j % NBUF == b)
            def _():
                pltpu.make_async_copy(
                    table_hbm.at[idx_v.at[0]], rows_v.at[b], gsems[b]
                ).wait()
                pltpu.async_copy(
                    rows_v.at[b], out_hbm.at[wid, :, j], wsems[b])

        @pl.when(j + NBUF < NCHUNK)
        def _():
            for b in range(NBUF):
                @pl.when(j % NBUF == b)
                def _():
                    pltpu.make_async_copy(
                        rows_v.at[b], out_hbm.at[wid, :, 0], wsems[b]
                    ).wait()
                    gather(j + NBUF, b)
        return 0

    lax.fori_loop(0, NCHUNK, chunk_body, 0)

    for b in range(NBUF):
        pltpu.make_async_copy(
            rows_v.at[b], out_hbm.at[wid, :, 0], wsems[b]
        ).wait()


def kernel(x, table):
    xt = x.T.astype(jnp.int32).reshape(HIST, NW, B_PER_W)
    out = _emb_lookup(xt, table)
    return out.reshape(BATCH, HIST, EMBED_DIM)


# speedup vs baseline: 1.0524x; 1.0524x over previous
import functools

import jax
import jax.numpy as jnp
from jax import lax
from jax.experimental import pallas as pl
from jax.experimental.pallas import tpu as pltpu
from jax.experimental.pallas import tpu_sc as plsc

VOCAB = 100000
EMBED_DIM = 64
BATCH = 4096
HIST = 50
N = BATCH * HIST

NUM_CORES = 2
NUM_SUBCORES = 16
NW = NUM_CORES * NUM_SUBCORES
B_PER_W = BATCH // NW
CHUNK = B_PER_W
NCHUNK = HIST
NBUF = 8

_mesh = plsc.VectorSubcoreMesh(core_axis_name="c", subcore_axis_name="s")


@functools.partial(
    pl.kernel,
    mesh=_mesh,
    out_type=jax.ShapeDtypeStruct((HIST, NW, B_PER_W, EMBED_DIM),
                                  jnp.float32),
    compiler_params=pltpu.CompilerParams(use_tc_tiling_on_sc=False),
    scratch_types=[
        pltpu.VMEM((NCHUNK, CHUNK), jnp.int32),
        pltpu.VMEM((NBUF, CHUNK, EMBED_DIM), jnp.float32),
    ] + [pltpu.SemaphoreType.DMA] * 16,
)
def _emb_lookup(x_hbm, table_hbm, out_hbm, idx_v, rows_v, *sems):
    wid = lax.axis_index("s") * NUM_CORES + lax.axis_index("c")

    pltpu.sync_copy(x_hbm.at[:, wid], idx_v)

    gsems = sems[:NBUF]
    wsems = sems[NBUF:]

    def gather(j, b):
        pltpu.async_copy(table_hbm.at[idx_v.at[j]], rows_v.at[b], gsems[b])

    for b in range(NBUF):
        gather(b, b)

    def chunk_body(j, _):
        for b in range(NBUF):
            @pl.when(j % NBUF == b)
            def _():
                pltpu.make_async_copy(
                    table_hbm.at[idx_v.at[0]], rows_v.at[b], gsems[b]
                ).wait()
                pltpu.async_copy(
                    rows_v.at[b], out_hbm.at[j, wid], wsems[b])

        @pl.when(j + NBUF < NCHUNK)
        def _():
            for b in range(NBUF):
                @pl.when(j % NBUF == b)
                def _():
                    pltpu.make_async_copy(
                        rows_v.at[b], out_hbm.at[0, wid], wsems[b]
                    ).wait()
                    gather(j + NBUF, b)
        return 0

    lax.fori_loop(0, NCHUNK, chunk_body, 0)

    for b in range(NBUF):
        pltpu.make_async_copy(
            rows_v.at[b], out_hbm.at[0, wid], wsems[b]
        ).wait()


def kernel(x, table):
    xt = x.T.astype(jnp.int32).reshape(HIST, NW, B_PER_W)
    out = _emb_lookup(xt, table)
    return out.reshape(HIST, BATCH, EMBED_DIM).transpose(1, 0, 2)
